# Initial kernel scaffold; baseline (speedup 1.0000x reference)
import jax
import jax.numpy as jnp
from jax.experimental import pallas as pl


def kernel(latent_token, pos_embed, Wg, bg, ln0_g, ln0_b, W1_0, b1_0, W2_0, b2_0, ln1_g, ln1_b, W1_1, b1_1, W2_1, b2_1, lnh_g, lnh_b, Wh, bh, adj_matrix):
    raise NotImplementedError("write your pallas kernel here")


# speedup vs baseline: 3.1797x; 3.1797x over previous
import functools

import jax
import jax.numpy as jnp
from jax import lax
from jax.experimental import pallas as pl
from jax.experimental.pallas import tpu as pltpu
from jax.experimental.pallas import tpu_sc as plsc

_B, _N, _D, _K = 4, 10000, 256, 6
_H = 4 * _D
_NPAD = 10240
_RTILE = 320
_TPB = _NPAD // _RTILE
_NW = 32
_CH = 64


def _make_gather6_sum(n_rows):
    rw = n_rows // _NW
    nchunk = rw // _CH
    mesh = plsc.VectorSubcoreMesh(core_axis_name="c", subcore_axis_name="s")

    @functools.partial(
        pl.kernel,
        mesh=mesh,
        out_type=jax.ShapeDtypeStruct((n_rows, _D), jnp.float32),
        scratch_types=[
            pltpu.VMEM((_K * rw,), jnp.int32),
            pltpu.VMEM((_K, _CH, _D), jnp.float32),
            pltpu.VMEM((_CH, _D), jnp.float32),
            pltpu.SemaphoreType.DMA,
        ],
    )
    def gather6(table_hbm, idx_hbm, out_hbm, idx_v, bufs_v, acc_v, sem):
        wid = lax.axis_index("s") * 2 + lax.axis_index("c")
        base = wid * rw
        for kk in range(_K):
            pltpu.sync_copy(idx_hbm.at[pl.ds(kk * n_rows + base, rw)],
                            idx_v.at[pl.ds(kk * rw, rw)])

        def chunk_body(c, carry):
            cps = [
                pltpu.async_copy(
                    table_hbm.at[idx_v.at[pl.ds(kk * rw + c * _CH, _CH)]],
                    bufs_v.at[kk], sem)
                for kk in range(_K)
            ]
            for cp in cps:
                cp.wait()

            def row_body(r, rcarry):
                for j in range(_D // 16):
                    s = pl.ds(j * 16, 16)
                    v = bufs_v[0, r, s]
                    for kk in range(1, _K):
                        v = v + bufs_v[kk, r, s]
                    acc_v[r, s] = v
                return rcarry

            lax.fori_loop(0, _CH, row_body, 0)
            pltpu.sync_copy(acc_v, out_hbm.at[pl.ds(base + c * _CH, _CH)])
            return carry

        lax.fori_loop(0, nchunk, chunk_body, 0)

    return gather6


def _gelu(x):
    return 0.5 * x * (1.0 + lax.erf(x * 0.7071067811865476))


def _ln(x, g, b):
    mu = jnp.mean(x, axis=-1, keepdims=True)
    xc = x - mu
    var = jnp.mean(xc * xc, axis=-1, keepdims=True)
    return xc * lax.rsqrt(var + 1e-5) * g + b


def _proj_body(lat_r, wg_r, bg_r, g_r):
    g_r[...] = (jnp.dot(lat_r[...], wg_r[...],
                        preferred_element_type=jnp.float32) + bg_r[...])


def _blk0_body(pos_r, psum_r, g_r, lng_r, lnb_r, w1_r, b1_r, w2_r, b2_r,
               h0_r):
    g = g_r[pl.ds(pl.program_id(0), 1), :]
    agg = g + (pos_r[...] + psum_r[...]) * (1.0 / 7.0)
    h = _ln(agg, lng_r[...], lnb_r[...])
    h = _gelu(
        jnp.dot(h, w1_r[...], preferred_element_type=jnp.float32) + b1_r[...])
    h0_r[...] = (jnp.dot(h, w2_r[...], preferred_element_type=jnp.float32)
                 + b2_r[...])


def _blk1_body(pos_r, psum_r, h0_r, hsum_r, g_r, lng_r, lnb_r, w1_r, b1_r,
               w2_r, b2_r, lnhg_r, lnhb_r, wh_r, bh_r, out_r):
    g = g_r[pl.ds(pl.program_id(0), 1), :]
    pos = pos_r[...]
    h0 = h0_r[...]
    agg = g + (pos + psum_r[...] + h0 + hsum_r[...]) * (1.0 / 7.0)
    h = _ln(agg, lng_r[...], lnb_r[...])
    h = _gelu(
        jnp.dot(h, w1_r[...], preferred_element_type=jnp.float32) + b1_r[...])
    h1 = (jnp.dot(h, w2_r[...], preferred_element_type=jnp.float32)
          + b2_r[...])
    nodes2 = g + pos + h0 + h1
    hn = _ln(nodes2, lnhg_r[...], lnhb_r[...])
    out_r[...] = (jnp.dot(hn, wh_r[...], preferred_element_type=jnp.float32)
                  + bh_r[...])


_gather_pos = _make_gather6_sum(_NPAD)
_gather_h0 = _make_gather6_sum(_B * _NPAD)


def kernel(latent_token, pos_embed, Wg, bg, ln0_g, ln0_b, W1_0, b1_0, W2_0,
           b2_0, ln1_g, ln1_b, W1_1, b1_1, W2_1, b2_1, lnh_g, lnh_b, Wh, bh,
           adj_matrix):
    pos_p = jnp.pad(pos_embed[0], ((0, _NPAD - _N), (0, 0)))
    adj_p = jnp.pad(adj_matrix, ((0, _NPAD - _N), (0, 0)))
    idx_a = adj_p.T
    offs = (jnp.arange(_B, dtype=jnp.int32) * _NPAD)[None, :, None]
    idx_b = (idx_a[:, None, :] + offs).reshape(_K, _B * _NPAD)
    lat_p = jnp.pad(latent_token, ((0, 8 - _B), (0, 0)))
    wh_p = jnp.pad(Wh, ((0, 0), (0, 6)))
    bh_p = jnp.pad(bh, (0, 6))

    g8 = pl.pallas_call(
        _proj_body,
        out_shape=jax.ShapeDtypeStruct((8, _D), jnp.float32),
    )(lat_p, Wg, bg)

    psum = _gather_pos(pos_p, idx_a.reshape(-1))

    full = lambda shape: pl.BlockSpec(shape, lambda b, t: (0,) * len(shape))
    row_in = pl.BlockSpec((_RTILE, _D), lambda b, t: (t, 0))
    row_bt = pl.BlockSpec((_RTILE, _D), lambda b, t: (b * _TPB + t, 0))
    g_spec = pl.BlockSpec((8, _D), lambda b, t: (0, 0))

    h0 = pl.pallas_call(
        _blk0_body,
        grid=(_B, _TPB),
        in_specs=[row_in, row_in, g_spec, full((_D,)), full((_D,)),
                  full((_D, _H)), full((_H,)), full((_H, _D)), full((_D,))],
        out_specs=row_bt,
        out_shape=jax.ShapeDtypeStruct((_B * _NPAD, _D), jnp.float32),
    )(pos_p, psum, g8, ln0_g, ln0_b, W1_0, b1_0, W2_0, b2_0)

    hsum = _gather_h0(h0, idx_b.reshape(-1))

    out = pl.pallas_call(
        _blk1_body,
        grid=(_B, _TPB),
        in_specs=[row_in, row_in, row_bt, row_bt, g_spec,
                  full((_D,)), full((_D,)), full((_D, _H)), full((_H,)),
                  full((_H, _D)), full((_D,)), full((_D,)), full((_D,)),
                  full((_D, 8)), full((8,))],
        out_specs=pl.BlockSpec((_RTILE, 8), lambda b, t: (b * _TPB + t, 0)),
        out_shape=jax.ShapeDtypeStruct((_B * _NPAD, 8), jnp.float32),
    )(pos_p, psum, h0, hsum, g8, ln1_g, ln1_b, W1_1, b1_1, W2_1, b2_1,
      lnh_g, lnh_b, wh_p, bh_p)

    out = out.reshape(_B, _NPAD, 8)[:, :_N, :2]
    return jnp.transpose(out, (0, 2, 1))
